# Initial kernel scaffold; baseline (speedup 1.0000x reference)
import functools

import jax
import jax.numpy as jnp
from jax import lax
from jax.experimental import pallas as pl
from jax.experimental.pallas import tpu as pltpu
from jax.experimental.pallas import tpu_sc as plsc

N_NODES = 10000
N_EDGES = 320000
D_FEAT = 128

NC = 2
NS = 16
NW = NC * NS
EPW = N_EDGES // NW
CHK = 80
NCHUNK = EPW // CHK
RPT = N_NODES // NS
DEGW = 16


def _sc_body(src_hbm, dst_hbm, hsrc_hbm, pacc_hbm, pdeg_hbm,
             sidx, didx, rows, ones, zbuf, zdeg, acc, deg, sem):
    cid = lax.axis_index("c")
    sid = lax.axis_index("s")
    wid = cid * NS + sid

    def _zrow(r, _):
        for c in range(D_FEAT // 16):
            zbuf[r, pl.ds(c * 16, 16)] = jnp.zeros((16,), jnp.float32)
        return 0
    lax.fori_loop(0, 125, _zrow, 0)

    def _zdrow(r, _):
        zdeg[r, pl.ds(0, 16)] = jnp.zeros((16,), jnp.float32)
        return 0
    lax.fori_loop(0, RPT, _zdrow, 0)

    def _orow(r, _):
        ones[r, pl.ds(0, 16)] = jnp.ones((16,), jnp.float32)
        return 0
    lax.fori_loop(0, CHK, _orow, 0)

    base_r = sid * RPT
    for j in range(RPT // 125):
        pltpu.sync_copy(zbuf, acc.at[pl.ds(base_r + j * 125, 125)])
    pltpu.sync_copy(zdeg, deg.at[pl.ds(base_r, RPT)])
    plsc.subcore_barrier()

    ebase = wid * EPW

    def _chunk(i, _):
        base = ebase + i * CHK
        pltpu.sync_copy(src_hbm.at[pl.ds(base, CHK)], sidx)
        pltpu.sync_copy(dst_hbm.at[pl.ds(base, CHK)], didx)
        pltpu.async_copy(hsrc_hbm.at[sidx], rows, sem).wait()
        pltpu.sync_copy(rows, acc.at[didx], add=True)
        pltpu.sync_copy(ones, deg.at[didx], add=True)
        return 0
    lax.fori_loop(0, NCHUNK, _chunk, 0)

    plsc.subcore_barrier()

    pltpu.sync_copy(acc.at[pl.ds(base_r, RPT)],
                    pacc_hbm.at[cid, pl.ds(base_r, RPT)])
    pltpu.sync_copy(deg.at[pl.ds(base_r, RPT)],
                    pdeg_hbm.at[cid, pl.ds(base_r, RPT)])


_sc_agg = functools.partial(
    pl.kernel,
    out_type=[
        jax.ShapeDtypeStruct((NC, N_NODES, D_FEAT), jnp.float32),
        jax.ShapeDtypeStruct((NC, N_NODES, DEGW), jnp.float32),
    ],
    mesh=plsc.VectorSubcoreMesh(core_axis_name="c", subcore_axis_name="s"),
    scratch_types=[
        pltpu.VMEM((CHK,), jnp.int32),
        pltpu.VMEM((CHK,), jnp.int32),
        pltpu.VMEM((CHK, D_FEAT), jnp.float32),
        pltpu.VMEM((CHK, DEGW), jnp.float32),
        pltpu.VMEM((125, D_FEAT), jnp.float32),
        pltpu.VMEM((RPT, DEGW), jnp.float32),
        pltpu.VMEM_SHARED((N_NODES, D_FEAT), jnp.float32),
        pltpu.VMEM_SHARED((N_NODES, DEGW), jnp.float32),
        pltpu.SemaphoreType.DMA,
    ],
)(_sc_body)


def _merge_body(p_ref, d_ref, hdst_ref, out_ref):
    p = p_ref[0] + p_ref[1]
    degc = d_ref[0, :, 0:1] + d_ref[1, :, 0:1]
    out_ref[...] = (p + hdst_ref[...]) / (degc + 1.0)


def _tc_merge(p, d, h_dst):
    blk = 1000
    grid = N_NODES // blk
    return pl.pallas_call(
        _merge_body,
        grid=(grid,),
        in_specs=[
            pl.BlockSpec((NC, blk, D_FEAT), lambda i: (0, i, 0)),
            pl.BlockSpec((NC, blk, DEGW), lambda i: (0, i, 0)),
            pl.BlockSpec((blk, D_FEAT), lambda i: (i, 0)),
        ],
        out_specs=pl.BlockSpec((blk, D_FEAT), lambda i: (i, 0)),
        out_shape=jax.ShapeDtypeStruct((N_NODES, D_FEAT), jnp.float32),
    )(p, d, h_dst)


@jax.jit
def kernel(h_src, h_dst, edge_index):
    src = edge_index[0]
    dst = edge_index[1]
    p, d = _sc_agg(src, dst, h_src)
    return _tc_merge(p, d, h_dst)


# speedup vs baseline: 6.2997x; 6.2997x over previous
import functools

import jax
import jax.numpy as jnp
from jax import lax
from jax.experimental import pallas as pl
from jax.experimental.pallas import tpu as pltpu
from jax.experimental.pallas import tpu_sc as plsc

N_NODES = 10000
N_EDGES = 320000
D_FEAT = 128

NC = 2
NS = 16
NW = NC * NS
EPW = N_EDGES // NW
CHK = 80
NCHUNK = EPW // CHK
NPAD = 10240
RPT = NPAD // NS
BLK = 1024


def _sc_body(src_hbm, dst_hbm, hsrc_hbm, pacc_hbm, pdeg_hbm,
             sidx, didx, rows, onesv, acc, deg, sem):
    cid = lax.axis_index("c")
    sid = lax.axis_index("s")
    wid = cid * NS + sid

    def _zrow(r, _):
        for c in range(D_FEAT // 16):
            rows[r, pl.ds(c * 16, 16)] = jnp.zeros((16,), jnp.float32)
        return 0
    lax.fori_loop(0, CHK, _zrow, 0)
    for c in range(CHK // 16):
        onesv[pl.ds(c * 16, 16)] = jnp.zeros((16,), jnp.float32)

    base_r = sid * RPT
    for j in range(RPT // CHK):
        pltpu.sync_copy(rows, acc.at[pl.ds(base_r + j * CHK, CHK)])
        pltpu.sync_copy(onesv, deg.at[pl.ds(base_r + j * CHK, CHK)])

    for c in range(CHK // 16):
        onesv[pl.ds(c * 16, 16)] = jnp.ones((16,), jnp.float32)
    plsc.subcore_barrier()

    ebase = wid * EPW

    def _chunk(i, _):
        base = ebase + i * CHK
        pltpu.sync_copy(src_hbm.at[pl.ds(base, CHK)], sidx)
        pltpu.sync_copy(dst_hbm.at[pl.ds(base, CHK)], didx)
        pltpu.async_copy(hsrc_hbm.at[sidx], rows, sem).wait()
        pltpu.sync_copy(rows, acc.at[didx], add=True)
        pltpu.sync_copy(onesv, deg.at[didx], add=True)
        return 0
    lax.fori_loop(0, NCHUNK, _chunk, 0)

    plsc.subcore_barrier()

    pltpu.sync_copy(acc.at[pl.ds(base_r, RPT)],
                    pacc_hbm.at[cid, pl.ds(base_r, RPT)])
    pltpu.sync_copy(deg.at[pl.ds(base_r, RPT)],
                    pdeg_hbm.at[cid, pl.ds(base_r, RPT)])


_sc_agg = functools.partial(
    pl.kernel,
    out_type=[
        jax.ShapeDtypeStruct((NC, NPAD, D_FEAT), jnp.float32),
        jax.ShapeDtypeStruct((NC, NPAD), jnp.float32),
    ],
    mesh=plsc.VectorSubcoreMesh(core_axis_name="c", subcore_axis_name="s"),
    scratch_types=[
        pltpu.VMEM((CHK,), jnp.int32),
        pltpu.VMEM((CHK,), jnp.int32),
        pltpu.VMEM((CHK, D_FEAT), jnp.float32),
        pltpu.VMEM((CHK,), jnp.float32),
        pltpu.VMEM_SHARED((NPAD, D_FEAT), jnp.float32),
        pltpu.VMEM_SHARED((NPAD,), jnp.float32),
        pltpu.SemaphoreType.DMA,
    ],
)(_sc_body)


def _merge_body(p_ref, d_ref, hdst_ref, out_ref):
    p = p_ref[0] + p_ref[1]
    degc = d_ref[0] + d_ref[1]
    out_ref[...] = (p + hdst_ref[...]) / (degc + 1.0)[:, None]


def _tc_merge(p, d, h_dst_pad):
    grid = NPAD // BLK
    return pl.pallas_call(
        _merge_body,
        grid=(grid,),
        in_specs=[
            pl.BlockSpec((NC, BLK, D_FEAT), lambda i: (0, i, 0)),
            pl.BlockSpec((NC, BLK), lambda i: (0, i)),
            pl.BlockSpec((BLK, D_FEAT), lambda i: (i, 0)),
        ],
        out_specs=pl.BlockSpec((BLK, D_FEAT), lambda i: (i, 0)),
        out_shape=jax.ShapeDtypeStruct((NPAD, D_FEAT), jnp.float32),
    )(p, d, h_dst_pad)


@jax.jit
def kernel(h_src, h_dst, edge_index):
    src = edge_index[0]
    dst = edge_index[1]
    p, d = _sc_agg(src, dst, h_src)
    h_dst_pad = jnp.pad(h_dst, ((0, NPAD - N_NODES), (0, 0)))
    out = _tc_merge(p, d, h_dst_pad)
    return out[:N_NODES]


# speedup vs baseline: 12.5492x; 1.9920x over previous
import functools

import jax
import jax.numpy as jnp
from jax import lax
from jax.experimental import pallas as pl
from jax.experimental.pallas import tpu as pltpu
from jax.experimental.pallas import tpu_sc as plsc

N_NODES = 10000
N_EDGES = 320000
D_FEAT = 128

NC = 2
NS = 16
NW = NC * NS
CHK = 128
NROWS = N_EDGES // CHK
IB = 8
NBLK = 10
ROWS2D = NW * NBLK * IB
NPAD = 10240
RPT = NPAD // NS
BLK = 1024


def _sc_body(src2d, dst2d, hsrc_hbm, pacc_hbm, pdeg_hbm,
             sidx, didx, rows0, rows1, onesv, acc, deg,
             sem_g, sem_a0, sem_a1, sem_d):
    cid = lax.axis_index("c")
    sid = lax.axis_index("s")
    wid = cid * NS + sid
    rowsb = (rows0, rows1)
    sema = (sem_a0, sem_a1)

    def _zrow(r, _):
        for c in range(D_FEAT // 16):
            rows0[r, pl.ds(c * 16, 16)] = jnp.zeros((16,), jnp.float32)
        return 0
    lax.fori_loop(0, CHK, _zrow, 0)
    for c in range(CHK // 16):
        onesv[pl.ds(c * 16, 16)] = jnp.zeros((16,), jnp.float32)

    base_r = sid * RPT
    for j in range(RPT // CHK):
        pltpu.sync_copy(rows0, acc.at[pl.ds(base_r + j * CHK, CHK)])
        pltpu.sync_copy(onesv, deg.at[pl.ds(base_r + j * CHK, CHK)])
    for c in range(CHK // 16):
        onesv[pl.ds(c * 16, 16)] = jnp.ones((16,), jnp.float32)
    plsc.subcore_barrier()

    row0 = wid * (NBLK * IB)

    def _block(blk, _):
        g0 = row0 + blk * IB
        pltpu.sync_copy(src2d.at[pl.ds(g0, IB)], sidx)
        pltpu.sync_copy(dst2d.at[pl.ds(g0, IB)], didx)
        for j in range(IB):
            b = j % 2
            if j >= 2:
                @pl.when(g0 + (j - 2) < NROWS)
                def _drain(j=j, b=b):
                    pltpu.make_async_copy(
                        rowsb[b], acc.at[didx.at[j - 2]], sema[b]).wait()
            @pl.when(g0 + j < NROWS)
            def _work(j=j, b=b):
                pltpu.async_copy(hsrc_hbm.at[sidx.at[j]], rowsb[b],
                                 sem_g).wait()
                pltpu.async_copy(rowsb[b], acc.at[didx.at[j]], sema[b],
                                 add=True)
                pltpu.async_copy(onesv, deg.at[didx.at[j]], sem_d, add=True)
        for j in (IB - 2, IB - 1):
            b = j % 2
            @pl.when(g0 + j < NROWS)
            def _drain2(j=j, b=b):
                pltpu.make_async_copy(
                    rowsb[b], acc.at[didx.at[j]], sema[b]).wait()
        for j in range(IB):
            @pl.when(g0 + j < NROWS)
            def _drain3(j=j):
                pltpu.make_async_copy(onesv, deg.at[didx.at[j]],
                                      sem_d).wait()
        return 0
    lax.fori_loop(0, NBLK, _block, 0)

    plsc.subcore_barrier()

    pltpu.sync_copy(acc.at[pl.ds(base_r, RPT)],
                    pacc_hbm.at[cid, pl.ds(base_r, RPT)])
    pltpu.sync_copy(deg.at[pl.ds(base_r, RPT)],
                    pdeg_hbm.at[cid, pl.ds(base_r, RPT)])


_sc_agg = functools.partial(
    pl.kernel,
    out_type=[
        jax.ShapeDtypeStruct((NC, NPAD, D_FEAT), jnp.float32),
        jax.ShapeDtypeStruct((NC, NPAD), jnp.float32),
    ],
    mesh=plsc.VectorSubcoreMesh(core_axis_name="c", subcore_axis_name="s"),
    scratch_types=[
        pltpu.VMEM((IB, CHK), jnp.int32),
        pltpu.VMEM((IB, CHK), jnp.int32),
        pltpu.VMEM((CHK, D_FEAT), jnp.float32),
        pltpu.VMEM((CHK, D_FEAT), jnp.float32),
        pltpu.VMEM((CHK,), jnp.float32),
        pltpu.VMEM_SHARED((NPAD, D_FEAT), jnp.float32),
        pltpu.VMEM_SHARED((NPAD,), jnp.float32),
        pltpu.SemaphoreType.DMA,
        pltpu.SemaphoreType.DMA,
        pltpu.SemaphoreType.DMA,
        pltpu.SemaphoreType.DMA,
    ],
)(_sc_body)


def _merge_body(p_ref, d_ref, hdst_ref, out_ref):
    p = p_ref[0] + p_ref[1]
    degc = d_ref[0] + d_ref[1]
    out_ref[...] = (p + hdst_ref[...]) / (degc + 1.0)[:, None]


def _tc_merge(p, d, h_dst_pad):
    grid = NPAD // BLK
    return pl.pallas_call(
        _merge_body,
        grid=(grid,),
        in_specs=[
            pl.BlockSpec((NC, BLK, D_FEAT), lambda i: (0, i, 0)),
            pl.BlockSpec((NC, BLK), lambda i: (0, i)),
            pl.BlockSpec((BLK, D_FEAT), lambda i: (i, 0)),
        ],
        out_specs=pl.BlockSpec((BLK, D_FEAT), lambda i: (i, 0)),
        out_shape=jax.ShapeDtypeStruct((NPAD, D_FEAT), jnp.float32),
    )(p, d, h_dst_pad)


@jax.jit
def kernel(h_src, h_dst, edge_index):
    e2d = jnp.pad(edge_index, ((0, 0), (0, ROWS2D * CHK - N_EDGES)))
    e2d = e2d.reshape(2, ROWS2D, CHK)
    p, d = _sc_agg(e2d[0], e2d[1], h_src)
    h_dst_pad = jnp.pad(h_dst, ((0, NPAD - N_NODES), (0, 0)))
    out = _tc_merge(p, d, h_dst_pad)
    return out[:N_NODES]


# speedup vs baseline: 13.4909x; 1.0750x over previous
import functools

import jax
import jax.numpy as jnp
from jax import lax
from jax.experimental import pallas as pl
from jax.experimental.pallas import tpu as pltpu
from jax.experimental.pallas import tpu_sc as plsc

N_NODES = 10000
N_EDGES = 320000
D_FEAT = 128

NC = 2
NS = 16
NW = NC * NS
CHK = 128
NROWS = N_EDGES // CHK
IB = 8
NBLK = 10
ROWS2D = NW * NBLK * IB
NPAD = 10240
RPT = NPAD // NS
BLK = 1024


def _sc_body(src2d, dst2d, hsrc_hbm, pacc_hbm, pdeg_hbm,
             sidx, didx, rows0, rows1, onesv, acc, deg,
             sem_g0, sem_g1, sem_a0, sem_a1, sem_d, sem_i):
    cid = lax.axis_index("c")
    sid = lax.axis_index("s")
    wid = cid * NS + sid
    rowsb = (rows0, rows1)
    sema = (sem_a0, sem_a1)
    semg = (sem_g0, sem_g1)

    def _zrow(r, _):
        for c in range(D_FEAT // 16):
            rows0[r, pl.ds(c * 16, 16)] = jnp.zeros((16,), jnp.float32)
        return 0
    lax.fori_loop(0, CHK, _zrow, 0)
    for c in range(CHK // 16):
        onesv[pl.ds(c * 16, 16)] = jnp.zeros((16,), jnp.float32)

    base_r = sid * RPT
    for j in range(RPT // CHK):
        pltpu.sync_copy(rows0, acc.at[pl.ds(base_r + j * CHK, CHK)])
        pltpu.sync_copy(onesv, deg.at[pl.ds(base_r + j * CHK, CHK)])
    for c in range(CHK // 16):
        onesv[pl.ds(c * 16, 16)] = jnp.ones((16,), jnp.float32)
    plsc.subcore_barrier()

    row0 = wid * (NBLK * IB)

    def _block(blk, _):
        g0 = row0 + blk * IB
        par = lax.rem(blk, 2)
        si = sidx.at[par]
        di = didx.at[par]
        pltpu.sync_copy(src2d.at[pl.ds(g0, IB)], si)
        pltpu.sync_copy(dst2d.at[pl.ds(g0, IB)], di)

        @pl.when(g0 < NROWS)
        def _prime():
            pltpu.async_copy(hsrc_hbm.at[si.at[0]], rows0, sem_g0)

        for j in range(IB):
            b = j % 2
            rb = rowsb[b]
            sg = semg[b]
            if j >= 1:
                @pl.when(g0 + j - 1 < NROWS)
                def _drain(j=j, rb2=rowsb[(j - 1) % 2], di=di,
                           sa=sema[(j - 1) % 2]):
                    pltpu.make_async_copy(rb2, acc.at[di.at[j - 1]],
                                          sa).wait()
            if j < IB - 1:
                @pl.when(g0 + j + 1 < NROWS)
                def _ahead(j=j, rb2=rowsb[(j + 1) % 2], si=si,
                           sg2=semg[(j + 1) % 2]):
                    pltpu.async_copy(hsrc_hbm.at[si.at[j + 1]], rb2, sg2)
            @pl.when(g0 + j < NROWS)
            def _work(j=j, b=b, rb=rb, sg=sg, si=si, di=di):
                pltpu.make_async_copy(hsrc_hbm.at[si.at[j]], rb, sg).wait()
                pltpu.async_copy(rb, acc.at[di.at[j]], sema[b], add=True)
                pltpu.async_copy(onesv, deg.at[di.at[j]], sem_d, add=True)

        @pl.when(g0 + IB - 1 < NROWS)
        def _drain_last():
            pltpu.make_async_copy(rowsb[(IB - 1) % 2],
                                  acc.at[di.at[IB - 1]],
                                  sema[(IB - 1) % 2]).wait()
        for j in range(IB):
            @pl.when(g0 + j < NROWS)
            def _drain3(j=j, di=di):
                pltpu.make_async_copy(onesv, deg.at[di.at[j]],
                                      sem_d).wait()
        return 0
    lax.fori_loop(0, NBLK, _block, 0)

    plsc.subcore_barrier()

    pltpu.sync_copy(acc.at[pl.ds(base_r, RPT)],
                    pacc_hbm.at[cid, pl.ds(base_r, RPT)])
    pltpu.sync_copy(deg.at[pl.ds(base_r, RPT)],
                    pdeg_hbm.at[cid, pl.ds(base_r, RPT)])


_sc_agg = functools.partial(
    pl.kernel,
    out_type=[
        jax.ShapeDtypeStruct((NC, NPAD, D_FEAT), jnp.float32),
        jax.ShapeDtypeStruct((NC, NPAD), jnp.float32),
    ],
    mesh=plsc.VectorSubcoreMesh(core_axis_name="c", subcore_axis_name="s"),
    scratch_types=[
        pltpu.VMEM((2, IB, CHK), jnp.int32),
        pltpu.VMEM((2, IB, CHK), jnp.int32),
        pltpu.VMEM((CHK, D_FEAT), jnp.float32),
        pltpu.VMEM((CHK, D_FEAT), jnp.float32),
        pltpu.VMEM((CHK,), jnp.float32),
        pltpu.VMEM_SHARED((NPAD, D_FEAT), jnp.float32),
        pltpu.VMEM_SHARED((NPAD,), jnp.float32),
        pltpu.SemaphoreType.DMA,
        pltpu.SemaphoreType.DMA,
        pltpu.SemaphoreType.DMA,
        pltpu.SemaphoreType.DMA,
        pltpu.SemaphoreType.DMA,
        pltpu.SemaphoreType.DMA,
    ],
)(_sc_body)


def _merge_body(p_ref, d_ref, hdst_ref, out_ref):
    p = p_ref[0] + p_ref[1]
    degc = d_ref[0] + d_ref[1]
    out_ref[...] = (p + hdst_ref[...]) / (degc + 1.0)[:, None]


def _tc_merge(p, d, h_dst_pad):
    grid = NPAD // BLK
    return pl.pallas_call(
        _merge_body,
        grid=(grid,),
        in_specs=[
            pl.BlockSpec((NC, BLK, D_FEAT), lambda i: (0, i, 0)),
            pl.BlockSpec((NC, BLK), lambda i: (0, i)),
            pl.BlockSpec((BLK, D_FEAT), lambda i: (i, 0)),
        ],
        out_specs=pl.BlockSpec((BLK, D_FEAT), lambda i: (i, 0)),
        out_shape=jax.ShapeDtypeStruct((NPAD, D_FEAT), jnp.float32),
    )(p, d, h_dst_pad)


@jax.jit
def kernel(h_src, h_dst, edge_index):
    e2d = jnp.pad(edge_index, ((0, 0), (0, ROWS2D * CHK - N_EDGES)))
    e2d = e2d.reshape(2, ROWS2D, CHK)
    p, d = _sc_agg(e2d[0], e2d[1], h_src)
    h_dst_pad = jnp.pad(h_dst, ((0, NPAD - N_NODES), (0, 0)))
    out = _tc_merge(p, d, h_dst_pad)
    return out[:N_NODES]


# speedup vs baseline: 14.4103x; 1.0681x over previous
import functools

import jax
import jax.numpy as jnp
from jax import lax
from jax.experimental import pallas as pl
from jax.experimental.pallas import tpu as pltpu
from jax.experimental.pallas import tpu_sc as plsc

N_NODES = 10000
N_EDGES = 320000
D_FEAT = 128

NC = 2
NS = 16
NW = NC * NS
CHK = 128
NROWS = N_EDGES // CHK
IB = 8
NBLK = 10
ROWS2D = NW * NBLK * IB
NPAD = 10240
RPT = NPAD // NS
BLK = 1024


def _sc_body(src2d, dst2d, hsrc_hbm, pacc_hbm, pdeg_hbm,
             sidx, didx, rows0, rows1, onesv, acc, deg,
             sem_g0, sem_g1, sem_a0, sem_a1, sem_d, sem_i):
    cid = lax.axis_index("c")
    sid = lax.axis_index("s")
    wid = cid * NS + sid
    rowsb = (rows0, rows1)
    sema = (sem_a0, sem_a1)
    semg = (sem_g0, sem_g1)

    def _zrow(r, _):
        for c in range(D_FEAT // 16):
            rows0[r, pl.ds(c * 16, 16)] = jnp.zeros((16,), jnp.float32)
        return 0
    lax.fori_loop(0, CHK, _zrow, 0)
    for c in range(CHK // 16):
        onesv[pl.ds(c * 16, 16)] = jnp.zeros((16,), jnp.float32)

    base_r = sid * RPT
    for j in range(RPT // CHK):
        pltpu.async_copy(rows0, acc.at[pl.ds(base_r + j * CHK, CHK)], sem_i)
        pltpu.async_copy(onesv, deg.at[pl.ds(base_r + j * CHK, CHK)], sem_i)
    for j in range(RPT // CHK):
        pltpu.make_async_copy(rows0, acc.at[pl.ds(base_r + j * CHK, CHK)],
                              sem_i).wait()
        pltpu.make_async_copy(onesv, deg.at[pl.ds(base_r + j * CHK, CHK)],
                              sem_i).wait()
    for c in range(CHK // 16):
        onesv[pl.ds(c * 16, 16)] = jnp.ones((16,), jnp.float32)
    plsc.subcore_barrier()

    row0 = wid * (NBLK * IB)

    def _block(blk, _):
        g0 = row0 + blk * IB
        sub = lax.rem(blk, NBLK // 2)
        lr0 = sub * IB

        @pl.when(sub == 0)
        def _loadidx():
            pltpu.sync_copy(src2d.at[pl.ds(g0, NBLK // 2 * IB)], sidx)
            pltpu.sync_copy(dst2d.at[pl.ds(g0, NBLK // 2 * IB)], didx)

        @pl.when(g0 < NROWS)
        def _prime():
            pltpu.async_copy(hsrc_hbm.at[sidx.at[lr0]], rows0, sem_g0)

        for j in range(IB):
            b = j % 2
            rb = rowsb[b]
            sg = semg[b]
            if j >= 1:
                @pl.when(g0 + j - 1 < NROWS)
                def _drain(j=j, rb2=rowsb[(j - 1) % 2],
                           sa=sema[(j - 1) % 2]):
                    pltpu.make_async_copy(rb2, acc.at[didx.at[lr0 + j - 1]],
                                          sa).wait()
            if j < IB - 1:
                @pl.when(g0 + j + 1 < NROWS)
                def _ahead(j=j, rb2=rowsb[(j + 1) % 2],
                           sg2=semg[(j + 1) % 2]):
                    pltpu.async_copy(hsrc_hbm.at[sidx.at[lr0 + j + 1]],
                                     rb2, sg2)
            @pl.when(g0 + j < NROWS)
            def _work(j=j, b=b, rb=rb, sg=sg):
                pltpu.make_async_copy(hsrc_hbm.at[sidx.at[lr0 + j]], rb,
                                      sg).wait()
                pltpu.async_copy(rb, acc.at[didx.at[lr0 + j]], sema[b],
                                 add=True)
                pltpu.async_copy(onesv, deg.at[didx.at[lr0 + j]], sem_d,
                                 add=True)

        @pl.when(g0 + IB - 1 < NROWS)
        def _drain_last():
            pltpu.make_async_copy(rowsb[(IB - 1) % 2],
                                  acc.at[didx.at[lr0 + IB - 1]],
                                  sema[(IB - 1) % 2]).wait()
        for j in range(IB):
            @pl.when(g0 + j < NROWS)
            def _drain3(j=j):
                pltpu.make_async_copy(onesv, deg.at[didx.at[lr0 + j]],
                                      sem_d).wait()
        return 0
    lax.fori_loop(0, NBLK, _block, 0)

    plsc.subcore_barrier()

    pltpu.sync_copy(acc.at[pl.ds(base_r, RPT)],
                    pacc_hbm.at[cid, pl.ds(base_r, RPT)])
    pltpu.sync_copy(deg.at[pl.ds(base_r, RPT)],
                    pdeg_hbm.at[cid, pl.ds(base_r, RPT)])


_sc_agg = functools.partial(
    pl.kernel,
    out_type=[
        jax.ShapeDtypeStruct((NC, NPAD, D_FEAT), jnp.float32),
        jax.ShapeDtypeStruct((NC, NPAD), jnp.float32),
    ],
    mesh=plsc.VectorSubcoreMesh(core_axis_name="c", subcore_axis_name="s"),
    scratch_types=[
        pltpu.VMEM((NBLK // 2 * IB, CHK), jnp.int32),
        pltpu.VMEM((NBLK // 2 * IB, CHK), jnp.int32),
        pltpu.VMEM((CHK, D_FEAT), jnp.float32),
        pltpu.VMEM((CHK, D_FEAT), jnp.float32),
        pltpu.VMEM((CHK,), jnp.float32),
        pltpu.VMEM_SHARED((NPAD, D_FEAT), jnp.float32),
        pltpu.VMEM_SHARED((NPAD,), jnp.float32),
        pltpu.SemaphoreType.DMA,
        pltpu.SemaphoreType.DMA,
        pltpu.SemaphoreType.DMA,
        pltpu.SemaphoreType.DMA,
        pltpu.SemaphoreType.DMA,
        pltpu.SemaphoreType.DMA,
    ],
)(_sc_body)


def _merge_body(p_ref, d_ref, hdst_ref, out_ref):
    p = p_ref[0] + p_ref[1]
    degc = d_ref[0] + d_ref[1]
    out_ref[...] = (p + hdst_ref[...]) / (degc + 1.0)[:, None]


def _tc_merge(p, d, h_dst_pad):
    grid = NPAD // BLK
    return pl.pallas_call(
        _merge_body,
        grid=(grid,),
        in_specs=[
            pl.BlockSpec((NC, BLK, D_FEAT), lambda i: (0, i, 0)),
            pl.BlockSpec((NC, BLK), lambda i: (0, i)),
            pl.BlockSpec((BLK, D_FEAT), lambda i: (i, 0)),
        ],
        out_specs=pl.BlockSpec((BLK, D_FEAT), lambda i: (i, 0)),
        out_shape=jax.ShapeDtypeStruct((NPAD, D_FEAT), jnp.float32),
    )(p, d, h_dst_pad)


@jax.jit
def kernel(h_src, h_dst, edge_index):
    e2d = jnp.pad(edge_index, ((0, 0), (0, ROWS2D * CHK - N_EDGES)))
    e2d = e2d.reshape(2, ROWS2D, CHK)
    p, d = _sc_agg(e2d[0], e2d[1], h_src)
    h_dst_pad = jnp.pad(h_dst, ((0, NPAD - N_NODES), (0, 0)))
    out = _tc_merge(p, d, h_dst_pad)
    return out[:N_NODES]


# speedup vs baseline: 15.4563x; 1.0726x over previous
import functools

import jax
import jax.numpy as jnp
from jax import lax
from jax.experimental import pallas as pl
from jax.experimental.pallas import tpu as pltpu
from jax.experimental.pallas import tpu_sc as plsc

N_NODES = 10000
N_EDGES = 320000
D_FEAT = 128

NC = 2
NS = 16
NW = NC * NS
CHK = 128
NROWS = N_EDGES // CHK
IB = 8
NBLK = 10
ROWS2D = NW * NBLK * IB
NPAD = 10240
RPT = NPAD // NS
BLK = 1024


def _sc_body(src2d, dst2d, hsrc_hbm, pacc_hbm, pdeg_hbm,
             sidx, didx, rows0, rows1, onesv, acc, deg,
             sem_g0, sem_g1, sem_a0, sem_a1, sem_d, sem_i):
    cid = lax.axis_index("c")
    sid = lax.axis_index("s")
    wid = cid * NS + sid
    rowsb = (rows0, rows1)
    sema = (sem_a0, sem_a1)
    semg = (sem_g0, sem_g1)

    def _zrow(r, _):
        for c in range(D_FEAT // 16):
            rows0[r, pl.ds(c * 16, 16)] = jnp.zeros((16,), jnp.float32)
        return 0
    lax.fori_loop(0, CHK, _zrow, 0)
    for c in range(CHK // 16):
        onesv[pl.ds(c * 16, 16)] = jnp.zeros((16,), jnp.float32)

    base_r = sid * RPT
    for j in range(RPT // CHK):
        pltpu.async_copy(rows0, acc.at[pl.ds(base_r + j * CHK, CHK)], sem_i)
        pltpu.async_copy(onesv, deg.at[pl.ds(base_r + j * CHK, CHK)], sem_i)
    for j in range(RPT // CHK):
        pltpu.make_async_copy(rows0, acc.at[pl.ds(base_r + j * CHK, CHK)],
                              sem_i).wait()
        pltpu.make_async_copy(onesv, deg.at[pl.ds(base_r + j * CHK, CHK)],
                              sem_i).wait()
    for c in range(CHK // 16):
        onesv[pl.ds(c * 16, 16)] = jnp.ones((16,), jnp.float32)
    plsc.subcore_barrier()

    row0 = wid * (NBLK * IB)

    HB = NBLK // 2 * IB

    def _block(blk, _):
        g0 = row0 + blk * IB
        sub = lax.rem(blk, NBLK // 2)
        lr0 = sub * IB
        half_g0 = g0 - lr0

        @pl.when(sub == 0)
        def _loadidx():
            pltpu.sync_copy(src2d.at[pl.ds(g0, HB)], sidx)
            pltpu.sync_copy(dst2d.at[pl.ds(g0, HB)], didx)

        @pl.when((sub == 0) & (g0 < NROWS))
        def _prime():
            pltpu.async_copy(hsrc_hbm.at[sidx.at[lr0]], rows0, sem_g0)

        for j in range(IB):
            b = j % 2
            rb = rowsb[b]
            sg = semg[b]
            if j == 0:
                @pl.when((sub > 0) & (g0 - 1 < NROWS))
                def _drain0():
                    pltpu.make_async_copy(rowsb[1], acc.at[didx.at[lr0 - 1]],
                                          sema[1]).wait()
            else:
                @pl.when(g0 + j - 1 < NROWS)
                def _drain(j=j, rb2=rowsb[(j - 1) % 2],
                           sa=sema[(j - 1) % 2]):
                    pltpu.make_async_copy(rb2, acc.at[didx.at[lr0 + j - 1]],
                                          sa).wait()
            if j < IB - 1:
                @pl.when(g0 + j + 1 < NROWS)
                def _ahead(j=j, rb2=rowsb[(j + 1) % 2],
                           sg2=semg[(j + 1) % 2]):
                    pltpu.async_copy(hsrc_hbm.at[sidx.at[lr0 + j + 1]],
                                     rb2, sg2)
            else:
                @pl.when((sub < NBLK // 2 - 1) & (g0 + IB < NROWS))
                def _ahead_x():
                    pltpu.async_copy(hsrc_hbm.at[sidx.at[lr0 + IB]],
                                     rows0, sem_g0)
            @pl.when(g0 + j < NROWS)
            def _work(j=j, b=b, rb=rb, sg=sg):
                pltpu.make_async_copy(hsrc_hbm.at[sidx.at[lr0 + j]], rb,
                                      sg).wait()
                pltpu.async_copy(rb, acc.at[didx.at[lr0 + j]], sema[b],
                                 add=True)
                pltpu.async_copy(onesv, deg.at[didx.at[lr0 + j]], sem_d,
                                 add=True)

        @pl.when(sub == NBLK // 2 - 1)
        def _flush():
            @pl.when(g0 + IB - 1 < NROWS)
            def _flush_acc():
                pltpu.make_async_copy(rowsb[(IB - 1) % 2],
                                      acc.at[didx.at[lr0 + IB - 1]],
                                      sema[(IB - 1) % 2]).wait()
            for lr in range(HB):
                @pl.when(half_g0 + lr < NROWS)
                def _flush_deg(lr=lr):
                    pltpu.make_async_copy(onesv, deg.at[didx.at[lr]],
                                          sem_d).wait()
        return 0
    lax.fori_loop(0, NBLK, _block, 0)

    plsc.subcore_barrier()

    pltpu.sync_copy(acc.at[pl.ds(base_r, RPT)],
                    pacc_hbm.at[cid, pl.ds(base_r, RPT)])
    pltpu.sync_copy(deg.at[pl.ds(base_r, RPT)],
                    pdeg_hbm.at[cid, pl.ds(base_r, RPT)])


_sc_agg = functools.partial(
    pl.kernel,
    out_type=[
        jax.ShapeDtypeStruct((NC, NPAD, D_FEAT), jnp.float32),
        jax.ShapeDtypeStruct((NC, NPAD), jnp.float32),
    ],
    mesh=plsc.VectorSubcoreMesh(core_axis_name="c", subcore_axis_name="s"),
    scratch_types=[
        pltpu.VMEM((NBLK // 2 * IB, CHK), jnp.int32),
        pltpu.VMEM((NBLK // 2 * IB, CHK), jnp.int32),
        pltpu.VMEM((CHK, D_FEAT), jnp.float32),
        pltpu.VMEM((CHK, D_FEAT), jnp.float32),
        pltpu.VMEM((CHK,), jnp.float32),
        pltpu.VMEM_SHARED((NPAD, D_FEAT), jnp.float32),
        pltpu.VMEM_SHARED((NPAD,), jnp.float32),
        pltpu.SemaphoreType.DMA,
        pltpu.SemaphoreType.DMA,
        pltpu.SemaphoreType.DMA,
        pltpu.SemaphoreType.DMA,
        pltpu.SemaphoreType.DMA,
        pltpu.SemaphoreType.DMA,
    ],
)(_sc_body)


def _merge_body(p_ref, d_ref, hdst_ref, out_ref):
    p = p_ref[0] + p_ref[1]
    degc = d_ref[0] + d_ref[1]
    out_ref[...] = (p + hdst_ref[...]) / (degc + 1.0)[:, None]


def _tc_merge(p, d, h_dst_pad):
    grid = NPAD // BLK
    return pl.pallas_call(
        _merge_body,
        grid=(grid,),
        in_specs=[
            pl.BlockSpec((NC, BLK, D_FEAT), lambda i: (0, i, 0)),
            pl.BlockSpec((NC, BLK), lambda i: (0, i)),
            pl.BlockSpec((BLK, D_FEAT), lambda i: (i, 0)),
        ],
        out_specs=pl.BlockSpec((BLK, D_FEAT), lambda i: (i, 0)),
        out_shape=jax.ShapeDtypeStruct((NPAD, D_FEAT), jnp.float32),
    )(p, d, h_dst_pad)


@jax.jit
def kernel(h_src, h_dst, edge_index):
    e2d = jnp.pad(edge_index, ((0, 0), (0, ROWS2D * CHK - N_EDGES)))
    e2d = e2d.reshape(2, ROWS2D, CHK)
    p, d = _sc_agg(e2d[0], e2d[1], h_src)
    h_dst_pad = jnp.pad(h_dst, ((0, NPAD - N_NODES), (0, 0)))
    out = _tc_merge(p, d, h_dst_pad)
    return out[:N_NODES]


# speedup vs baseline: 15.6984x; 1.0157x over previous
import functools

import jax
import jax.numpy as jnp
from jax import lax
from jax.experimental import pallas as pl
from jax.experimental.pallas import tpu as pltpu
from jax.experimental.pallas import tpu_sc as plsc

N_NODES = 10000
N_EDGES = 320000
D_FEAT = 128

NC = 2
NS = 16
NW = NC * NS
CHK = 128
NROWS = N_EDGES // CHK
IB = 8
NBLK = 10
ROWS2D = NW * NBLK * IB
NPAD = 10240
RPT = NPAD // NS
BLK = 1024


def _sc_body(src2d, dst2d, hsrc_hbm, pacc_hbm, pdeg_hbm,
             sidx, didx, rows0, rows1, onesv, acc, deg,
             sem_g0, sem_g1, sem_a0, sem_a1, sem_d, sem_i, sem_i2):
    cid = lax.axis_index("c")
    sid = lax.axis_index("s")
    wid = cid * NS + sid
    rowsb = (rows0, rows1)
    sema = (sem_a0, sem_a1)
    semg = (sem_g0, sem_g1)
    row0 = wid * (NBLK * IB)
    HB = NBLK // 2 * IB

    pltpu.async_copy(src2d.at[pl.ds(row0, HB)], sidx, sem_i2)
    pltpu.async_copy(dst2d.at[pl.ds(row0, HB)], didx, sem_i2)

    def _zrow(r, _):
        for c in range(D_FEAT // 16):
            rows0[r, pl.ds(c * 16, 16)] = jnp.zeros((16,), jnp.float32)
        return 0
    lax.fori_loop(0, CHK, _zrow, 0)
    for c in range(CHK // 16):
        onesv[pl.ds(c * 16, 16)] = jnp.zeros((16,), jnp.float32)

    base_r = sid * RPT
    for j in range(RPT // CHK):
        pltpu.async_copy(rows0, acc.at[pl.ds(base_r + j * CHK, CHK)], sem_i)
        pltpu.async_copy(onesv, deg.at[pl.ds(base_r + j * CHK, CHK)], sem_i)
    for j in range(RPT // CHK):
        pltpu.make_async_copy(rows0, acc.at[pl.ds(base_r + j * CHK, CHK)],
                              sem_i).wait()
        pltpu.make_async_copy(onesv, deg.at[pl.ds(base_r + j * CHK, CHK)],
                              sem_i).wait()
    for c in range(CHK // 16):
        onesv[pl.ds(c * 16, 16)] = jnp.ones((16,), jnp.float32)
    plsc.subcore_barrier()

    def _block(blk, _):
        g0 = row0 + blk * IB
        sub = lax.rem(blk, NBLK // 2)
        lr0 = sub * IB
        half_g0 = g0 - lr0

        @pl.when((sub == 0) & (blk > 0))
        def _loadidx():
            pltpu.sync_copy(src2d.at[pl.ds(g0, HB)], sidx)
            pltpu.sync_copy(dst2d.at[pl.ds(g0, HB)], didx)

        @pl.when((sub == 0) & (blk == 0))
        def _waitidx():
            pltpu.make_async_copy(src2d.at[pl.ds(row0, HB)], sidx,
                                  sem_i2).wait()
            pltpu.make_async_copy(dst2d.at[pl.ds(row0, HB)], didx,
                                  sem_i2).wait()

        @pl.when((sub == 0) & (g0 < NROWS))
        def _prime():
            pltpu.async_copy(hsrc_hbm.at[sidx.at[lr0]], rows0, sem_g0)

        for j in range(IB):
            b = j % 2
            rb = rowsb[b]
            sg = semg[b]
            if j == 0:
                @pl.when((sub > 0) & (g0 - 1 < NROWS))
                def _drain0():
                    pltpu.make_async_copy(rowsb[1], acc.at[didx.at[lr0 - 1]],
                                          sema[1]).wait()
            else:
                @pl.when(g0 + j - 1 < NROWS)
                def _drain(j=j, rb2=rowsb[(j - 1) % 2],
                           sa=sema[(j - 1) % 2]):
                    pltpu.make_async_copy(rb2, acc.at[didx.at[lr0 + j - 1]],
                                          sa).wait()
            if j < IB - 1:
                @pl.when(g0 + j + 1 < NROWS)
                def _ahead(j=j, rb2=rowsb[(j + 1) % 2],
                           sg2=semg[(j + 1) % 2]):
                    pltpu.async_copy(hsrc_hbm.at[sidx.at[lr0 + j + 1]],
                                     rb2, sg2)
            else:
                @pl.when((sub < NBLK // 2 - 1) & (g0 + IB < NROWS))
                def _ahead_x():
                    pltpu.async_copy(hsrc_hbm.at[sidx.at[lr0 + IB]],
                                     rows0, sem_g0)
            @pl.when(g0 + j < NROWS)
            def _work(j=j, b=b, rb=rb, sg=sg):
                pltpu.make_async_copy(hsrc_hbm.at[sidx.at[lr0 + j]], rb,
                                      sg).wait()
                pltpu.async_copy(rb, acc.at[didx.at[lr0 + j]], sema[b],
                                 add=True)
                pltpu.async_copy(onesv, deg.at[didx.at[lr0 + j]], sem_d,
                                 add=True)

        @pl.when(sub == NBLK // 2 - 1)
        def _flush():
            @pl.when(g0 + IB - 1 < NROWS)
            def _flush_acc():
                pltpu.make_async_copy(rowsb[(IB - 1) % 2],
                                      acc.at[didx.at[lr0 + IB - 1]],
                                      sema[(IB - 1) % 2]).wait()
            for lr in range(HB):
                @pl.when(half_g0 + lr < NROWS)
                def _flush_deg(lr=lr):
                    pltpu.make_async_copy(onesv, deg.at[didx.at[lr]],
                                          sem_d).wait()
        return 0
    lax.fori_loop(0, NBLK, _block, 0)

    plsc.subcore_barrier()

    pltpu.async_copy(acc.at[pl.ds(base_r, RPT)],
                     pacc_hbm.at[cid, pl.ds(base_r, RPT)], sem_i)
    pltpu.async_copy(deg.at[pl.ds(base_r, RPT)],
                     pdeg_hbm.at[cid, pl.ds(base_r, RPT)], sem_i2)
    pltpu.make_async_copy(acc.at[pl.ds(base_r, RPT)],
                          pacc_hbm.at[cid, pl.ds(base_r, RPT)], sem_i).wait()
    pltpu.make_async_copy(deg.at[pl.ds(base_r, RPT)],
                          pdeg_hbm.at[cid, pl.ds(base_r, RPT)],
                          sem_i2).wait()


_sc_agg = functools.partial(
    pl.kernel,
    out_type=[
        jax.ShapeDtypeStruct((NC, NPAD, D_FEAT), jnp.float32),
        jax.ShapeDtypeStruct((NC, NPAD), jnp.float32),
    ],
    mesh=plsc.VectorSubcoreMesh(core_axis_name="c", subcore_axis_name="s"),
    scratch_types=[
        pltpu.VMEM((NBLK // 2 * IB, CHK), jnp.int32),
        pltpu.VMEM((NBLK // 2 * IB, CHK), jnp.int32),
        pltpu.VMEM((CHK, D_FEAT), jnp.float32),
        pltpu.VMEM((CHK, D_FEAT), jnp.float32),
        pltpu.VMEM((CHK,), jnp.float32),
        pltpu.VMEM_SHARED((NPAD, D_FEAT), jnp.float32),
        pltpu.VMEM_SHARED((NPAD,), jnp.float32),
        pltpu.SemaphoreType.DMA,
        pltpu.SemaphoreType.DMA,
        pltpu.SemaphoreType.DMA,
        pltpu.SemaphoreType.DMA,
        pltpu.SemaphoreType.DMA,
        pltpu.SemaphoreType.DMA,
        pltpu.SemaphoreType.DMA,
    ],
)(_sc_body)


def _merge_body(p_ref, d_ref, hdst_ref, out_ref):
    p = p_ref[0] + p_ref[1]
    degc = d_ref[0] + d_ref[1]
    out_ref[...] = (p + hdst_ref[...]) / (degc + 1.0)[:, None]


def _tc_merge(p, d, h_dst_pad):
    grid = NPAD // BLK
    return pl.pallas_call(
        _merge_body,
        grid=(grid,),
        in_specs=[
            pl.BlockSpec((NC, BLK, D_FEAT), lambda i: (0, i, 0)),
            pl.BlockSpec((NC, BLK), lambda i: (0, i)),
            pl.BlockSpec((BLK, D_FEAT), lambda i: (i, 0)),
        ],
        out_specs=pl.BlockSpec((BLK, D_FEAT), lambda i: (i, 0)),
        out_shape=jax.ShapeDtypeStruct((NPAD, D_FEAT), jnp.float32),
    )(p, d, h_dst_pad)


@jax.jit
def kernel(h_src, h_dst, edge_index):
    e2d = jnp.pad(edge_index, ((0, 0), (0, ROWS2D * CHK - N_EDGES)))
    e2d = e2d.reshape(2, ROWS2D, CHK)
    p, d = _sc_agg(e2d[0], e2d[1], h_src)
    h_dst_pad = jnp.pad(h_dst, ((0, NPAD - N_NODES), (0, 0)))
    out = _tc_merge(p, d, h_dst_pad)
    return out[:N_NODES]
